# baseline (speedup 1.0000x reference)
import functools

import jax
import jax.numpy as jnp
from jax import lax
from jax.experimental import pallas as pl
from jax.experimental.pallas import tpu as pltpu
from jax.experimental.pallas import tpu_sc as plsc

_NCLS = 8
_GATHER_DNUMS = lax.GatherDimensionNumbers(
    offset_dims=(), collapsed_slice_dims=(0,), start_index_map=(0,))
_NC = 2
_NS = 16
_NW = _NC * _NS
_L = 16
_R = 512


def _make_sc_kernel(n, d):
    rows_per_w = n // _NW
    nchunks = rows_per_w // _R
    mesh = plsc.VectorSubcoreMesh(core_axis_name="c", subcore_axis_name="s")

    @functools.partial(
        pl.kernel,
        out_type=jax.ShapeDtypeStruct((n, d), jnp.float32),
        mesh=mesh,
        scratch_types=[
            pltpu.VMEM((_NCLS * d,), jnp.float32),
            pltpu.VMEM((_NCLS * d,), jnp.float32),
            pltpu.VMEM((_R, d), jnp.float32),
            pltpu.VMEM((_R, d), jnp.float32),
            pltpu.VMEM((_R, d), jnp.float32),
            pltpu.VMEM((_R, d), jnp.float32),
            pltpu.VMEM((_R,), jnp.int32),
            pltpu.VMEM((_R,), jnp.int32),
            pltpu.SemaphoreType.DMA,
            pltpu.SemaphoreType.DMA,
            pltpu.SemaphoreType.DMA,
            pltpu.SemaphoreType.DMA,
        ],
        compiler_params=pltpu.CompilerParams(
            needs_layout_passes=False, use_tc_tiling_on_sc=False),
    )
    def sc_kernel(x_hbm, lab_hbm, scale_hbm, shift_hbm, out_hbm,
                  sc_v, sh_v, xb0, xb1, ob0, ob1, lb0, lb1,
                  lsem0, lsem1, ssem0, ssem1):
        xb = (xb0, xb1)
        ob = (ob0, ob1)
        lb = (lb0, lb1)
        lsem = (lsem0, lsem1)
        ssem = (ssem0, ssem1)

        wid = lax.axis_index("s") * _NC + lax.axis_index("c")
        base = wid * rows_per_w

        pltpu.sync_copy(scale_hbm, sc_v)
        pltpu.sync_copy(shift_hbm, sh_v)

        def start_load(row0, b):
            pltpu.make_async_copy(
                x_hbm.at[pl.ds(row0, _R), :], xb[b], lsem[b]).start()
            pltpu.make_async_copy(
                lab_hbm.at[pl.ds(row0, _R)], lb[b], lsem[b]).start()

        def wait_load(b):
            pltpu.make_async_copy(
                x_hbm.at[pl.ds(0, _R), :], xb[b], lsem[b]).wait()
            pltpu.make_async_copy(
                lab_hbm.at[pl.ds(0, _R)], lb[b], lsem[b]).wait()

        def start_store(row0, b):
            pltpu.make_async_copy(
                ob[b], out_hbm.at[pl.ds(row0, _R), :], ssem[b]).start()

        def wait_store(b):
            pltpu.make_async_copy(
                ob[b], out_hbm.at[pl.ds(0, _R), :], ssem[b]).wait()

        cols0 = lax.iota(jnp.int32, _L)
        cols1 = cols0 + _L

        def compute(b):
            def group(g, carry):
                for j in range(_L):
                    r = g * _L + j
                    bc = plsc.load_gather(
                        lb[b], [jnp.full((_L,), r, jnp.int32)]) << 5
                    i0 = bc + cols0
                    i1 = bc + cols1
                    s0 = plsc.load_gather(sc_v, [i0])
                    s1 = plsc.load_gather(sc_v, [i1])
                    t0 = plsc.load_gather(sh_v, [i0])
                    t1 = plsc.load_gather(sh_v, [i1])
                    x0 = xb[b][r, pl.ds(0, _L)]
                    x1 = xb[b][r, pl.ds(_L, _L)]
                    ob[b][r, pl.ds(0, _L)] = x0 * s0 + t0
                    ob[b][r, pl.ds(_L, _L)] = x1 * s1 + t1
                return carry
            lax.fori_loop(0, _R // _L, group, 0)

        start_load(base, 0)

        def outer(c2, carry):
            for b in range(2):
                cc = c2 * 2 + b
                nxt = jnp.minimum(base + (cc + 1) * _R, n - _R)
                start_load(nxt, 1 - b)
                wait_load(b)
                @pl.when(cc >= 2)
                def _():
                    wait_store(b)
                compute(b)
                start_store(base + cc * _R, b)
            return carry

        lax.fori_loop(0, nchunks // 2, outer, 0)

        wait_load(0)
        wait_store(0)
        wait_store(1)

    return sc_kernel


def kernel(x, labels, scale, shift):
    n, d = x.shape
    sck = _make_sc_kernel(n, d)
    return sck(x, labels.astype(jnp.int32), scale.reshape(-1),
               shift.reshape(-1))


# speedup vs baseline: 6.0186x; 6.0186x over previous
import jax
import jax.numpy as jnp
from jax import lax
from jax.experimental import pallas as pl
from jax.experimental.pallas import tpu as pltpu

_NCLS = 8
_BLK = 8192


def _body(lab_ref, scale_ref, shift_ref, x_ref, o_ref):
    lab = lab_ref[...]
    iot = lax.broadcasted_iota(jnp.int32, (_NCLS, 1), 0)
    oh = (lab == iot).astype(jnp.float32)
    rs = jnp.dot(scale_ref[...], oh, preferred_element_type=jnp.float32,
                 precision=lax.Precision.HIGHEST)
    rb = jnp.dot(shift_ref[...], oh, preferred_element_type=jnp.float32,
                 precision=lax.Precision.HIGHEST)
    o_ref[...] = x_ref[...] * rs + rb


def kernel(x, labels, scale, shift):
    n, d = x.shape
    xt = jnp.swapaxes(x, 0, 1)
    lab2 = labels.reshape(1, n)
    st = jnp.swapaxes(scale, 0, 1)
    tt = jnp.swapaxes(shift, 0, 1)
    grid = (n // _BLK,)
    out_t = pl.pallas_call(
        _body,
        grid=grid,
        in_specs=[
            pl.BlockSpec((1, _BLK), lambda i: (0, i)),
            pl.BlockSpec((d, _NCLS), lambda i: (0, 0)),
            pl.BlockSpec((d, _NCLS), lambda i: (0, 0)),
            pl.BlockSpec((d, _BLK), lambda i: (0, i)),
        ],
        out_specs=pl.BlockSpec((d, _BLK), lambda i: (0, i)),
        out_shape=jax.ShapeDtypeStruct((d, n), x.dtype),
        compiler_params=pltpu.CompilerParams(
            dimension_semantics=("arbitrary",),
        ),
    )(lab2, st, tt, xt)
    return jnp.swapaxes(out_t, 0, 1)


# speedup vs baseline: 13.3361x; 2.2158x over previous
import jax
import jax.numpy as jnp
from jax import lax
from jax.experimental import pallas as pl
from jax.experimental.pallas import tpu as pltpu

_NCLS = 8
_BLK = 16384


def _body(lab_ref, scale_ref, shift_ref, x_ref, o_ref):
    lab = lab_ref[...]
    iot = lax.broadcasted_iota(jnp.int32, (_NCLS, 1), 0)
    oh = (lab == iot).astype(jnp.float32)
    rs = jnp.dot(scale_ref[...], oh, preferred_element_type=jnp.float32)
    rb = jnp.dot(shift_ref[...], oh, preferred_element_type=jnp.float32)
    o_ref[...] = x_ref[...] * rs + rb


def kernel(x, labels, scale, shift):
    n, d = x.shape
    xt = jnp.swapaxes(x, 0, 1)
    lab2 = labels.reshape(1, n)
    st = jnp.swapaxes(scale, 0, 1)
    tt = jnp.swapaxes(shift, 0, 1)
    grid = (n // _BLK,)
    out_t = pl.pallas_call(
        _body,
        grid=grid,
        in_specs=[
            pl.BlockSpec((1, _BLK), lambda i: (0, i)),
            pl.BlockSpec((d, _NCLS), lambda i: (0, 0)),
            pl.BlockSpec((d, _NCLS), lambda i: (0, 0)),
            pl.BlockSpec((d, _BLK), lambda i: (0, i)),
        ],
        out_specs=pl.BlockSpec((d, _BLK), lambda i: (0, i)),
        out_shape=jax.ShapeDtypeStruct((d, n), x.dtype),
        compiler_params=pltpu.CompilerParams(
            dimension_semantics=("arbitrary",),
        ),
    )(lab2, st, tt, xt)
    return jnp.swapaxes(out_t, 0, 1)
